# baseline (speedup 1.0000x reference)
import functools

import jax
import jax.numpy as jnp
from jax import lax
from jax.experimental import pallas as pl
from jax.experimental.pallas import tpu as pltpu
from jax.experimental.pallas import tpu_sc as plsc

_N = 5000
_NP = 5120
_W = _NP // 32
_NC = 2
_NS = 16
_NW = _NC * _NS
_CHUNK_ROWS = 16
_NCHUNKS = _NP // _CHUNK_ROWS
_T = _NCHUNKS // _NW
_R2 = 64
_IOU = 0.25

_mesh = plsc.VectorSubcoreMesh(core_axis_name="c", subcore_axis_name="s")


def _p1_body(p7_hbm, m_hbm, p7_v, buf_v):
    cax = lax.axis_index("c")
    sax = lax.axis_index("s")
    wid = sax * _NC + cax
    pltpu.sync_copy(p7_hbm, p7_v)
    l16 = lax.iota(jnp.int32, 16)
    l160 = l16 * _W
    zz = jnp.zeros((16,), jnp.int32)

    def chunk_body(t, _):
        chunk = t * _NW + wid
        c0 = chunk * _CHUNK_ROWS
        jl = c0 + l16
        lozj = p7_v[pl.ds(0 * _NP + c0, 16)]
        loyj = p7_v[pl.ds(1 * _NP + c0, 16)]
        loxj = p7_v[pl.ds(2 * _NP + c0, 16)]
        hizj = p7_v[pl.ds(3 * _NP + c0, 16)]
        hiyj = p7_v[pl.ds(4 * _NP + c0, 16)]
        hixj = p7_v[pl.ds(5 * _NP + c0, 16)]
        volj = p7_v[pl.ds(6 * _NP + c0, 16)]
        for g in range(_CHUNK_ROWS * _W // 16):
            buf_v[pl.ds(g * 16, 16)] = zz
        w_start = c0 // 32

        z16 = jnp.zeros((16,), jnp.int32)

        def word_eval(w, masked):
            acc = jnp.zeros((16,), jnp.int32)
            for h in range(2):
                base = w * 32 + h * 16
                vecs = [p7_v[pl.ds(a * _NP + base, 16)] for a in range(4)]
                for dl in range(16):
                    di = h * 16 + dl
                    i = base + dl
                    lozi = vecs[0][dl]
                    loyi = vecs[1][dl]
                    loxi = vecs[2][dl]
                    hizi = vecs[3][dl]
                    hiyi = plsc.load_gather(p7_v, [z16 + (4 * _NP + i)])
                    hixi = plsc.load_gather(p7_v, [z16 + (5 * _NP + i)])
                    voli = plsc.load_gather(p7_v, [z16 + (6 * _NP + i)])
                    dz = jnp.maximum(jnp.minimum(hizj, hizi) - jnp.maximum(lozj, lozi), 0.0)
                    dy = jnp.maximum(jnp.minimum(hiyj, hiyi) - jnp.maximum(loyj, loyi), 0.0)
                    dx = jnp.maximum(jnp.minimum(hixj, hixi) - jnp.maximum(loxj, loxi), 0.0)
                    ov = (dz * dy) * dx
                    un = (voli + volj) - ov
                    hit = ov > _IOU * un
                    if masked:
                        hit = hit & (i > jl)
                    bitval = jnp.int32(-2147483648) if di == 31 else jnp.int32(1 << di)
                    acc = acc | jnp.where(hit, bitval, jnp.int32(0))
            plsc.store_scatter(buf_v, [l160 + w], acc)

        word_eval(w_start, True)

        def wbody(w, carry):
            word_eval(w, False)
            return carry

        lax.fori_loop(w_start + 1, _W, wbody, 0)
        pltpu.sync_copy(buf_v, m_hbm.at[pl.ds(c0 * _W, _CHUNK_ROWS * _W)])
        return _

    lax.fori_loop(0, _T, chunk_body, 0)


def _p2_body(m_hbm, o7_hbm, out_hbm, mbuf_v, rem_v, io_v, msp, s0, s1, s2, s3):
    cax = lax.axis_index("c")
    sax = lax.axis_index("s")
    wid = sax * _NC + cax
    l16 = lax.iota(jnp.int32, 16)
    sems = (s0, s1, s2, s3)
    _CH = 32
    _CHW = _CH * _W
    _NB = 4
    _NCH = _NP // _CH

    _SLICE = _NP * _W // _NS
    pltpu.sync_copy(
        m_hbm.at[pl.ds(sax * _SLICE, _SLICE)],
        msp.at[pl.ds(sax * _SLICE, _SLICE)],
    )
    plsc.subcore_barrier()

    @pl.when(wid == 0)
    def _():
        pltpu.sync_copy(o7_hbm, io_v)
        for b in range(_NB):
            pltpu.async_copy(
                msp.at[pl.ds(b * _CHW, _CHW)],
                mbuf_v.at[pl.ds(b * _CHW, _CHW)],
                sems[b],
            )

        zv = jnp.zeros((16,), jnp.int32)
        init = (zv,) * 10

        def super_body(q, rem):
            for b in range(_NB):
                cc = q * _NB + b
                pltpu.make_async_copy(
                    msp.at[pl.ds(0, _CHW)],
                    mbuf_v.at[pl.ds(b * _CHW, _CHW)],
                    sems[b],
                ).wait()
                for k in range(10):
                    rem_v[pl.ds(k * 16, 16)] = rem[k]
                z16 = jnp.zeros((16,), jnp.int32)
                localv = plsc.load_gather(rem_v, [z16 + cc])
                rem = list(rem)
                cur = [mbuf_v[pl.ds(b * _CHW + k * 16, 16)] for k in range(10)]
                dwc = plsc.load_gather(mbuf_v, [z16 + (b * _CHW + cc)])
                for r in range(_CH):
                    if r + 1 < _CH:
                        nro = b * _CHW + (r + 1) * _W
                        nxt = [mbuf_v[pl.ds(nro + k * 16, 16)] for k in range(10)]
                        dwn = plsc.load_gather(mbuf_v, [z16 + (nro + cc)])
                    bitc = jnp.int32(-2147483648) if r == 31 else jnp.int32(1 << r)
                    imsk = jnp.where((localv & bitc) == 0, jnp.int32(-1), jnp.int32(0))
                    localv = localv | (dwc & imsk)
                    for k in range(10):
                        rem[k] = rem[k] | (cur[k] & imsk)
                    if r + 1 < _CH:
                        cur, dwc = nxt, dwn
                rem = tuple(rem)

                @pl.when(cc + _NB < _NCH)
                def _():
                    pltpu.async_copy(
                        msp.at[pl.ds((cc + _NB) * _CHW, _CHW)],
                        mbuf_v.at[pl.ds(b * _CHW, _CHW)],
                        sems[b],
                    )

            return rem

        rem = lax.fori_loop(0, _NCH // _NB, super_body, init)
        for k in range(10):
            rem_v[pl.ds(k * 16, 16)] = rem[k]

        def mask_body(v, carry):
            word = rem_v[pl.ds(v // 2, 16)][0]
            sh = l16 + (v & 1) * 16
            bits = (word >> sh) & 1
            keep = 1.0 - bits.astype(jnp.float32)
            for a in range(7):
                off = a * _NP + v * 16
                io_v[pl.ds(off, 16)] = io_v[pl.ds(off, 16)] * keep
            return carry

        lax.fori_loop(0, _NP // 16, mask_body, 0)
        pltpu.sync_copy(io_v, out_hbm)


_phase1 = functools.partial(
    pl.kernel,
    out_type=jax.ShapeDtypeStruct((_NP * _W,), jnp.int32),
    mesh=_mesh,
    scratch_types=[
        pltpu.VMEM((7 * _NP,), jnp.float32),
        pltpu.VMEM((_CHUNK_ROWS * _W,), jnp.int32),
    ],
    compiler_params=pltpu.CompilerParams(needs_layout_passes=False),
)(_p1_body)

_phase2 = functools.partial(
    pl.kernel,
    out_type=jax.ShapeDtypeStruct((7 * _NP,), jnp.float32),
    mesh=_mesh,
    scratch_types=[
        pltpu.VMEM((4 * 32 * _W + 192,), jnp.int32),
        pltpu.VMEM((_W + 16,), jnp.int32),
        pltpu.VMEM((7 * _NP,), jnp.float32),
        pltpu.VMEM_SHARED((_NP * _W,), jnp.int32),
        pltpu.SemaphoreType.DMA,
        pltpu.SemaphoreType.DMA,
        pltpu.SemaphoreType.DMA,
        pltpu.SemaphoreType.DMA,
    ],
    compiler_params=pltpu.CompilerParams(needs_layout_passes=False),
)(_p2_body)


def kernel(boxes, scores):
    order = jnp.argsort(-scores)
    b = jnp.take(boxes, order, axis=0)
    s = jnp.take(scores, order)
    c = b[:, :3]
    sz = b[:, 3:]
    lo = c - sz / 2
    hi = c + sz / 2
    vol = jnp.prod(sz, axis=-1)
    pad = _NP - _N
    far = jnp.full((pad,), 1e9, jnp.float32)
    zpad = jnp.zeros((pad,), jnp.float32)

    def padcat(x, p):
        return jnp.concatenate([x, p])

    p7 = jnp.concatenate([
        padcat(lo[:, 0], far), padcat(lo[:, 1], far), padcat(lo[:, 2], far),
        padcat(hi[:, 0], far), padcat(hi[:, 1], far), padcat(hi[:, 2], far),
        padcat(vol, zpad),
    ])
    o7 = jnp.concatenate([
        padcat(s, zpad),
        padcat(b[:, 0], zpad), padcat(b[:, 1], zpad), padcat(b[:, 2], zpad),
        padcat(b[:, 3], zpad), padcat(b[:, 4], zpad), padcat(b[:, 5], zpad),
    ])
    m = _phase1(p7)
    out7 = _phase2(m, o7)
    return out7.reshape(7, _NP).T[:_N]


# speedup vs baseline: 1.0810x; 1.0810x over previous
import functools

import jax
import jax.numpy as jnp
from jax import lax
from jax.experimental import pallas as pl
from jax.experimental.pallas import tpu as pltpu
from jax.experimental.pallas import tpu_sc as plsc

_N = 5000
_NP = 5120
_W = _NP // 32
_NC = 2
_NS = 16
_NW = _NC * _NS
_CHUNK_ROWS = 16
_NCHUNKS = _NP // _CHUNK_ROWS
_T = _NCHUNKS // _NW
_R2 = 64
_IOU = 0.25

_mesh = plsc.VectorSubcoreMesh(core_axis_name="c", subcore_axis_name="s")


def _p1_body(p7_hbm, m_hbm, p7_v, buf_v):
    cax = lax.axis_index("c")
    sax = lax.axis_index("s")
    wid = sax * _NC + cax
    pltpu.sync_copy(p7_hbm, p7_v)
    l16 = lax.iota(jnp.int32, 16)
    l160 = l16 * _W
    zz = jnp.zeros((16,), jnp.int32)

    def chunk_body(t, _):
        chunk = t * _NW + wid
        c0 = chunk * _CHUNK_ROWS
        jl = c0 + l16
        lozj = p7_v[pl.ds(0 * _NP + c0, 16)]
        loyj = p7_v[pl.ds(1 * _NP + c0, 16)]
        loxj = p7_v[pl.ds(2 * _NP + c0, 16)]
        hizj = p7_v[pl.ds(3 * _NP + c0, 16)]
        hiyj = p7_v[pl.ds(4 * _NP + c0, 16)]
        hixj = p7_v[pl.ds(5 * _NP + c0, 16)]
        volj = p7_v[pl.ds(6 * _NP + c0, 16)]
        for g in range(_CHUNK_ROWS * _W // 16):
            buf_v[pl.ds(g * 16, 16)] = zz
        w_start = c0 // 32

        def word_eval(w, masked):
            acc = jnp.zeros((16,), jnp.int32)
            for h in range(2):
                base = w * 32 + h * 16
                vecs = [p7_v[pl.ds(a * _NP + base, 16)] for a in range(7)]
                for dl in range(16):
                    di = h * 16 + dl
                    i = base + dl
                    lozi = vecs[0][dl]
                    loyi = vecs[1][dl]
                    loxi = vecs[2][dl]
                    hizi = vecs[3][dl]
                    hiyi = vecs[4][dl]
                    hixi = vecs[5][dl]
                    voli = vecs[6][dl]
                    dz = jnp.maximum(jnp.minimum(hizj, hizi) - jnp.maximum(lozj, lozi), 0.0)
                    dy = jnp.maximum(jnp.minimum(hiyj, hiyi) - jnp.maximum(loyj, loyi), 0.0)
                    dx = jnp.maximum(jnp.minimum(hixj, hixi) - jnp.maximum(loxj, loxi), 0.0)
                    ov = (dz * dy) * dx
                    un = (voli + volj) - ov
                    hit = ov > _IOU * un
                    if masked:
                        hit = hit & (i > jl)
                    bitval = jnp.int32(-2147483648) if di == 31 else jnp.int32(1 << di)
                    acc = acc | jnp.where(hit, bitval, jnp.int32(0))
            plsc.store_scatter(buf_v, [l160 + w], acc)

        word_eval(w_start, True)

        def wbody(w, carry):
            word_eval(w, False)
            return carry

        lax.fori_loop(w_start + 1, _W, wbody, 0)
        pltpu.sync_copy(buf_v, m_hbm.at[pl.ds(c0 * _W, _CHUNK_ROWS * _W)])
        return _

    lax.fori_loop(0, _T, chunk_body, 0)


def _p2_body(m_hbm, o7_hbm, out_hbm, mbuf_v, rem_v, io_v, s0, s1, s2, s3):
    cax = lax.axis_index("c")
    sax = lax.axis_index("s")
    wid = sax * _NC + cax
    l16 = lax.iota(jnp.int32, 16)
    sems = (s0, s1, s2, s3)
    _CH = 32
    _CHW = _CH * _W
    _NB = 4
    _NCH = _NP // _CH

    @pl.when(wid == 0)
    def _():
        pltpu.sync_copy(o7_hbm, io_v)
        for b in range(_NB):
            pltpu.async_copy(
                m_hbm.at[pl.ds(b * _CHW, _CHW)],
                mbuf_v.at[pl.ds(b * _CHW, _CHW)],
                sems[b],
            )

        zv = jnp.zeros((16,), jnp.int32)
        init = (zv,) * 10

        def super_body(q, rem):
            for b in range(_NB):
                cc = q * _NB + b
                pltpu.make_async_copy(
                    m_hbm.at[pl.ds(0, _CHW)],
                    mbuf_v.at[pl.ds(b * _CHW, _CHW)],
                    sems[b],
                ).wait()
                for k in range(10):
                    rem_v[pl.ds(k * 16, 16)] = rem[k]
                z16 = jnp.zeros((16,), jnp.int32)
                localv = plsc.load_gather(rem_v, [z16 + cc])
                for r in range(_CH):
                    dwv = plsc.load_gather(
                        mbuf_v, [z16 + (b * _CHW + r * _W + cc)])
                    bitc = jnp.int32(-2147483648) if r == 31 else jnp.int32(1 << r)
                    imsk = jnp.where((localv & bitc) == 0, jnp.int32(-1), jnp.int32(0))
                    localv = localv | (dwv & imsk)
                rem = list(rem)
                for half in range(4):
                    msks = []
                    for rr in range(8):
                        r = half * 8 + rr
                        bitc = (jnp.int32(-2147483648) if r == 31
                                else jnp.int32(1 << r))
                        msks.append((localv & bitc) == 0)
                    for k in range(10):
                        acc = rem[k]
                        for rr in range(8):
                            r = half * 8 + rr
                            mv = mbuf_v[pl.ds(b * _CHW + r * _W + k * 16, 16)]
                            acc = acc | jnp.where(msks[rr], mv, zv)
                        rem[k] = acc
                rem = tuple(rem)

                @pl.when(cc + _NB < _NCH)
                def _():
                    pltpu.async_copy(
                        m_hbm.at[pl.ds((cc + _NB) * _CHW, _CHW)],
                        mbuf_v.at[pl.ds(b * _CHW, _CHW)],
                        sems[b],
                    )

            return rem

        rem = lax.fori_loop(0, _NCH // _NB, super_body, init)
        for k in range(10):
            rem_v[pl.ds(k * 16, 16)] = rem[k]

        def mask_body(v, carry):
            word = rem_v[pl.ds(v // 2, 16)][0]
            sh = l16 + (v & 1) * 16
            bits = (word >> sh) & 1
            keep = 1.0 - bits.astype(jnp.float32)
            for a in range(7):
                off = a * _NP + v * 16
                io_v[pl.ds(off, 16)] = io_v[pl.ds(off, 16)] * keep
            return carry

        lax.fori_loop(0, _NP // 16, mask_body, 0)
        pltpu.sync_copy(io_v, out_hbm)


_phase1 = functools.partial(
    pl.kernel,
    out_type=jax.ShapeDtypeStruct((_NP * _W,), jnp.int32),
    mesh=_mesh,
    scratch_types=[
        pltpu.VMEM((7 * _NP,), jnp.float32),
        pltpu.VMEM((_CHUNK_ROWS * _W,), jnp.int32),
    ],
    compiler_params=pltpu.CompilerParams(needs_layout_passes=False),
)(_p1_body)

_phase2 = functools.partial(
    pl.kernel,
    out_type=jax.ShapeDtypeStruct((7 * _NP,), jnp.float32),
    mesh=_mesh,
    scratch_types=[
        pltpu.VMEM((4 * 32 * _W + 192,), jnp.int32),
        pltpu.VMEM((_W + 16,), jnp.int32),
        pltpu.VMEM((7 * _NP,), jnp.float32),
        pltpu.SemaphoreType.DMA,
        pltpu.SemaphoreType.DMA,
        pltpu.SemaphoreType.DMA,
        pltpu.SemaphoreType.DMA,
    ],
    compiler_params=pltpu.CompilerParams(needs_layout_passes=False),
)(_p2_body)


def kernel(boxes, scores):
    order = jnp.argsort(-scores)
    b = jnp.take(boxes, order, axis=0)
    s = jnp.take(scores, order)
    c = b[:, :3]
    sz = b[:, 3:]
    lo = c - sz / 2
    hi = c + sz / 2
    vol = jnp.prod(sz, axis=-1)
    pad = _NP - _N
    far = jnp.full((pad,), 1e9, jnp.float32)
    zpad = jnp.zeros((pad,), jnp.float32)

    def padcat(x, p):
        return jnp.concatenate([x, p])

    p7 = jnp.concatenate([
        padcat(lo[:, 0], far), padcat(lo[:, 1], far), padcat(lo[:, 2], far),
        padcat(hi[:, 0], far), padcat(hi[:, 1], far), padcat(hi[:, 2], far),
        padcat(vol, zpad),
    ])
    o7 = jnp.concatenate([
        padcat(s, zpad),
        padcat(b[:, 0], zpad), padcat(b[:, 1], zpad), padcat(b[:, 2], zpad),
        padcat(b[:, 3], zpad), padcat(b[:, 4], zpad), padcat(b[:, 5], zpad),
    ])
    m = _phase1(p7)
    out7 = _phase2(m, o7)
    return out7.reshape(7, _NP).T[:_N]


# speedup vs baseline: 1.1847x; 1.0959x over previous
import functools

import jax
import jax.numpy as jnp
from jax import lax
from jax.experimental import pallas as pl
from jax.experimental.pallas import tpu as pltpu
from jax.experimental.pallas import tpu_sc as plsc

_N = 5000
_NP = 5120
_W = _NP // 32
_NC = 2
_NS = 16
_NW = _NC * _NS
_CHUNK_ROWS = 16
_NCHUNKS = _NP // _CHUNK_ROWS
_T = _NCHUNKS // _NW
_R2 = 64
_IOU = 0.25

_mesh = plsc.VectorSubcoreMesh(core_axis_name="c", subcore_axis_name="s")


_NG = _NP // 32
_GPW = _NG // _NW


def _p1_body(p7_hbm, m_hbm, p7_v, buf_v):
    cax = lax.axis_index("c")
    sax = lax.axis_index("s")
    wid = sax * _NC + cax
    pltpu.sync_copy(p7_hbm, p7_v)
    l16 = lax.iota(jnp.int32, 16)
    l160 = l16 * _W
    zz = jnp.zeros((16,), jnp.int32)

    def group_body(t, _):
        g_fwd = t * _NW + wid
        g_rev = (t + 1) * _NW - 1 - wid
        g = jnp.where(t % 2 == 0, g_fwd, g_rev)
        c0 = g * 32
        jlA = c0 + l16
        jlB = c0 + 16 + l16
        jA = [p7_v[pl.ds(a * _NP + c0, 16)] for a in range(7)]
        jB = [p7_v[pl.ds(a * _NP + c0 + 16, 16)] for a in range(7)]
        for q in range(32 * _W // 16):
            buf_v[pl.ds(q * 16, 16)] = zz

        def word_eval(w, masked):
            accA = jnp.zeros((16,), jnp.int32)
            accB = jnp.zeros((16,), jnp.int32)
            for h in range(2):
                base = w * 32 + h * 16
                vecs = [p7_v[pl.ds(a * _NP + base, 16)] for a in range(7)]
                for dl in range(16):
                    di = h * 16 + dl
                    i = base + dl
                    b7 = [vecs[a][dl] for a in range(7)]
                    bitval = jnp.int32(-2147483648) if di == 31 else jnp.int32(1 << di)
                    for side in range(2):
                        jv = jA if side == 0 else jB
                        jl = jlA if side == 0 else jlB
                        dz = jnp.maximum(
                            jnp.minimum(jv[3], b7[3]) - jnp.maximum(jv[0], b7[0]), 0.0)
                        dy = jnp.maximum(
                            jnp.minimum(jv[4], b7[4]) - jnp.maximum(jv[1], b7[1]), 0.0)
                        dx = jnp.maximum(
                            jnp.minimum(jv[5], b7[5]) - jnp.maximum(jv[2], b7[2]), 0.0)
                        ov = (dz * dy) * dx
                        un = (b7[6] + jv[6]) - ov
                        hit = ov > _IOU * un
                        if masked:
                            hit = hit & (i > jl)
                        bits = jnp.where(hit, bitval, jnp.int32(0))
                        if side == 0:
                            accA = accA | bits
                        else:
                            accB = accB | bits
            plsc.store_scatter(buf_v, [l160 + w], accA)
            plsc.store_scatter(buf_v, [l160 + 16 * _W + w], accB)

        word_eval(g, True)

        def wbody(w, carry):
            word_eval(w, False)
            return carry

        lax.fori_loop(g + 1, _W, wbody, 0)
        pltpu.sync_copy(buf_v, m_hbm.at[pl.ds(c0 * _W, 32 * _W)])
        return _

    lax.fori_loop(0, _GPW, group_body, 0)


def _p2_body(m_hbm, o7_hbm, out_hbm, mbuf_v, rem_v, io_v, s0, s1, s2, s3):
    cax = lax.axis_index("c")
    sax = lax.axis_index("s")
    wid = sax * _NC + cax
    l16 = lax.iota(jnp.int32, 16)
    sems = (s0, s1, s2, s3)
    _CH = 32
    _CHW = _CH * _W
    _NB = 4
    _NCH = _NP // _CH

    @pl.when(wid == 0)
    def _():
        pltpu.sync_copy(o7_hbm, io_v)
        for b in range(_NB):
            pltpu.async_copy(
                m_hbm.at[pl.ds(b * _CHW, _CHW)],
                mbuf_v.at[pl.ds(b * _CHW, _CHW)],
                sems[b],
            )

        zv = jnp.zeros((16,), jnp.int32)
        init = (zv,) * 10

        def super_body(q, rem):
            for b in range(_NB):
                cc = q * _NB + b
                pltpu.make_async_copy(
                    m_hbm.at[pl.ds(0, _CHW)],
                    mbuf_v.at[pl.ds(b * _CHW, _CHW)],
                    sems[b],
                ).wait()
                for k in range(10):
                    rem_v[pl.ds(k * 16, 16)] = rem[k]
                z16 = jnp.zeros((16,), jnp.int32)
                localv = plsc.load_gather(rem_v, [z16 + cc])
                for r in range(_CH):
                    dwv = plsc.load_gather(
                        mbuf_v, [z16 + (b * _CHW + r * _W + cc)])
                    bitc = jnp.int32(-2147483648) if r == 31 else jnp.int32(1 << r)
                    imsk = jnp.where((localv & bitc) == 0, jnp.int32(-1), jnp.int32(0))
                    localv = localv | (dwv & imsk)
                rem = list(rem)
                for half in range(4):
                    msks = []
                    for rr in range(8):
                        r = half * 8 + rr
                        bitc = (jnp.int32(-2147483648) if r == 31
                                else jnp.int32(1 << r))
                        msks.append((localv & bitc) == 0)
                    for k in range(10):
                        acc = rem[k]
                        for rr in range(8):
                            r = half * 8 + rr
                            mv = mbuf_v[pl.ds(b * _CHW + r * _W + k * 16, 16)]
                            acc = acc | jnp.where(msks[rr], mv, zv)
                        rem[k] = acc
                rem = tuple(rem)

                @pl.when(cc + _NB < _NCH)
                def _():
                    pltpu.async_copy(
                        m_hbm.at[pl.ds((cc + _NB) * _CHW, _CHW)],
                        mbuf_v.at[pl.ds(b * _CHW, _CHW)],
                        sems[b],
                    )

            return rem

        rem = lax.fori_loop(0, _NCH // _NB, super_body, init)
        for k in range(10):
            rem_v[pl.ds(k * 16, 16)] = rem[k]

        def mask_body(v, carry):
            word = rem_v[pl.ds(v // 2, 16)][0]
            sh = l16 + (v & 1) * 16
            bits = (word >> sh) & 1
            keep = 1.0 - bits.astype(jnp.float32)
            for a in range(7):
                off = a * _NP + v * 16
                io_v[pl.ds(off, 16)] = io_v[pl.ds(off, 16)] * keep
            return carry

        lax.fori_loop(0, _NP // 16, mask_body, 0)
        pltpu.sync_copy(io_v, out_hbm)


_phase1 = functools.partial(
    pl.kernel,
    out_type=jax.ShapeDtypeStruct((_NP * _W,), jnp.int32),
    mesh=_mesh,
    scratch_types=[
        pltpu.VMEM((7 * _NP,), jnp.float32),
        pltpu.VMEM((32 * _W,), jnp.int32),
    ],
    compiler_params=pltpu.CompilerParams(needs_layout_passes=False),
)(_p1_body)

_phase2 = functools.partial(
    pl.kernel,
    out_type=jax.ShapeDtypeStruct((7 * _NP,), jnp.float32),
    mesh=_mesh,
    scratch_types=[
        pltpu.VMEM((4 * 32 * _W + 192,), jnp.int32),
        pltpu.VMEM((_W + 16,), jnp.int32),
        pltpu.VMEM((7 * _NP,), jnp.float32),
        pltpu.SemaphoreType.DMA,
        pltpu.SemaphoreType.DMA,
        pltpu.SemaphoreType.DMA,
        pltpu.SemaphoreType.DMA,
    ],
    compiler_params=pltpu.CompilerParams(needs_layout_passes=False),
)(_p2_body)


def kernel(boxes, scores):
    order = jnp.argsort(-scores)
    b = jnp.take(boxes, order, axis=0)
    s = jnp.take(scores, order)
    c = b[:, :3]
    sz = b[:, 3:]
    lo = c - sz / 2
    hi = c + sz / 2
    vol = jnp.prod(sz, axis=-1)
    pad = _NP - _N
    far = jnp.full((pad,), 1e9, jnp.float32)
    zpad = jnp.zeros((pad,), jnp.float32)

    def padcat(x, p):
        return jnp.concatenate([x, p])

    p7 = jnp.concatenate([
        padcat(lo[:, 0], far), padcat(lo[:, 1], far), padcat(lo[:, 2], far),
        padcat(hi[:, 0], far), padcat(hi[:, 1], far), padcat(hi[:, 2], far),
        padcat(vol, zpad),
    ])
    o7 = jnp.concatenate([
        padcat(s, zpad),
        padcat(b[:, 0], zpad), padcat(b[:, 1], zpad), padcat(b[:, 2], zpad),
        padcat(b[:, 3], zpad), padcat(b[:, 4], zpad), padcat(b[:, 5], zpad),
    ])
    m = _phase1(p7)
    out7 = _phase2(m, o7)
    return out7.reshape(7, _NP).T[:_N]


# speedup vs baseline: 1.4078x; 1.1884x over previous
import functools

import jax
import jax.numpy as jnp
from jax import lax
from jax.experimental import pallas as pl
from jax.experimental.pallas import tpu as pltpu
from jax.experimental.pallas import tpu_sc as plsc

_N = 5000
_NP = 5120
_W = _NP // 32
_NC = 2
_NS = 16
_NW = _NC * _NS
_CHUNK_ROWS = 16
_NCHUNKS = _NP // _CHUNK_ROWS
_T = _NCHUNKS // _NW
_R2 = 64
_IOU = 0.25

_mesh = plsc.VectorSubcoreMesh(core_axis_name="c", subcore_axis_name="s")


_NG = _NP // 32
_GPW = _NG // _NW


def _p1_body(p7_hbm, m_hbm, p7_v, buf_v):
    cax = lax.axis_index("c")
    sax = lax.axis_index("s")
    wid = sax * _NC + cax
    pltpu.sync_copy(p7_hbm, p7_v)
    l16 = lax.iota(jnp.int32, 16)
    l160 = l16 * _W
    zz = jnp.zeros((16,), jnp.int32)

    def group_body(t, _):
        g_fwd = t * _NW + wid
        g_rev = (t + 1) * _NW - 1 - wid
        g = jnp.where(t % 2 == 0, g_fwd, g_rev)
        c0 = g * 32
        jlA = c0 + l16
        jlB = c0 + 16 + l16
        jA = [p7_v[pl.ds(a * _NP + c0, 16)] for a in range(7)]
        jB = [p7_v[pl.ds(a * _NP + c0 + 16, 16)] for a in range(7)]
        for q in range(32 * _W // 16):
            buf_v[pl.ds(q * 16, 16)] = zz

        def word_eval(w, masked):
            accA = jnp.zeros((16,), jnp.int32)
            accB = jnp.zeros((16,), jnp.int32)
            for h in range(2):
                base = w * 32 + h * 16
                vecs = [p7_v[pl.ds(a * _NP + base, 16)] for a in range(7)]
                for dl in range(16):
                    di = h * 16 + dl
                    i = base + dl
                    b7 = [vecs[a][dl] for a in range(7)]
                    bitval = jnp.int32(-2147483648) if di == 31 else jnp.int32(1 << di)
                    for side in range(2):
                        jv = jA if side == 0 else jB
                        jl = jlA if side == 0 else jlB
                        dz = jnp.maximum(
                            jnp.minimum(jv[3], b7[3]) - jnp.maximum(jv[0], b7[0]), 0.0)
                        dy = jnp.maximum(
                            jnp.minimum(jv[4], b7[4]) - jnp.maximum(jv[1], b7[1]), 0.0)
                        dx = jnp.maximum(
                            jnp.minimum(jv[5], b7[5]) - jnp.maximum(jv[2], b7[2]), 0.0)
                        ov = (dz * dy) * dx
                        un = (b7[6] + jv[6]) - ov
                        hit = ov > _IOU * un
                        if masked:
                            hit = hit & (i > jl)
                        bits = jnp.where(hit, bitval, jnp.int32(0))
                        if side == 0:
                            accA = accA | bits
                        else:
                            accB = accB | bits
            plsc.store_scatter(buf_v, [l160 + w], accA)
            plsc.store_scatter(buf_v, [l160 + 16 * _W + w], accB)

        word_eval(g, True)

        def wbody(w, carry):
            word_eval(w, False)
            return carry

        lax.fori_loop(g + 1, _W, wbody, 0)
        pltpu.sync_copy(buf_v, m_hbm.at[pl.ds(c0 * _W, 32 * _W)])
        return _

    lax.fori_loop(0, _GPW, group_body, 0)


def _p2_body(m_hbm, o7_hbm, out_hbm, mbuf_v, rem_v, io_v, s0, s1, s2, s3, s4):
    cax = lax.axis_index("c")
    sax = lax.axis_index("s")
    wid = sax * _NC + cax
    l16 = lax.iota(jnp.int32, 16)
    sems = (s0, s1, s2, s3)
    _CH = 32
    _CHW = _CH * _W
    _NB = 4
    _NCH = _NP // _CH

    @pl.when(wid == 0)
    def _():
        io_cp = pltpu.async_copy(o7_hbm, io_v, s4)
        zv = jnp.zeros((16,), jnp.int32)
        for k in range(11):
            rem_v[pl.ds(k * 16, 16)] = zv
        for b in range(_NB):
            pltpu.async_copy(
                m_hbm.at[pl.ds(b * _CHW, _CHW)],
                mbuf_v.at[pl.ds(b * _CHW, _CHW)],
                sems[b],
            )

        def super_body(q, carry):
            for b in range(_NB):
                cc = q * _NB + b
                pltpu.make_async_copy(
                    m_hbm.at[pl.ds(0, _CHW)],
                    mbuf_v.at[pl.ds(b * _CHW, _CHW)],
                    sems[b],
                ).wait()
                z16 = jnp.zeros((16,), jnp.int32)
                localv = plsc.load_gather(rem_v, [z16 + cc])
                for r in range(_CH):
                    dwv = plsc.load_gather(
                        mbuf_v, [z16 + (b * _CHW + r * _W + cc)])
                    bitc = jnp.int32(-2147483648) if r == 31 else jnp.int32(1 << r)
                    imsk = jnp.where((localv & bitc) == 0, jnp.int32(-1), jnp.int32(0))
                    localv = localv | (dwv & imsk)
                for half in range(4):
                    msks = []
                    for rr in range(8):
                        r = half * 8 + rr
                        bitc = (jnp.int32(-2147483648) if r == 31
                                else jnp.int32(1 << r))
                        msks.append((localv & bitc) == 0)
                    for k in range(10):
                        acc = rem_v[pl.ds(k * 16, 16)]
                        for rr in range(8):
                            r = half * 8 + rr
                            mv = mbuf_v[pl.ds(b * _CHW + r * _W + k * 16, 16)]
                            acc = acc | jnp.where(msks[rr], mv, zv)
                        rem_v[pl.ds(k * 16, 16)] = acc
                    del msks

                @pl.when(cc + _NB < _NCH)
                def _():
                    pltpu.async_copy(
                        m_hbm.at[pl.ds((cc + _NB) * _CHW, _CHW)],
                        mbuf_v.at[pl.ds(b * _CHW, _CHW)],
                        sems[b],
                    )

            return carry

        lax.fori_loop(0, _NCH // _NB, super_body, 0)
        io_cp.wait()

        def mask_body(v, carry):
            word = rem_v[pl.ds(v // 2, 16)][0]
            sh = l16 + (v & 1) * 16
            bits = (word >> sh) & 1
            keep = 1.0 - bits.astype(jnp.float32)
            for a in range(7):
                off = a * _NP + v * 16
                io_v[pl.ds(off, 16)] = io_v[pl.ds(off, 16)] * keep
            return carry

        lax.fori_loop(0, _NP // 16, mask_body, 0)
        pltpu.sync_copy(io_v, out_hbm)


_phase1 = functools.partial(
    pl.kernel,
    out_type=jax.ShapeDtypeStruct((_NP * _W,), jnp.int32),
    mesh=_mesh,
    scratch_types=[
        pltpu.VMEM((7 * _NP,), jnp.float32),
        pltpu.VMEM((32 * _W,), jnp.int32),
    ],
    compiler_params=pltpu.CompilerParams(needs_layout_passes=False),
)(_p1_body)

_phase2 = functools.partial(
    pl.kernel,
    out_type=jax.ShapeDtypeStruct((7 * _NP,), jnp.float32),
    mesh=_mesh,
    scratch_types=[
        pltpu.VMEM((4 * 32 * _W + 192,), jnp.int32),
        pltpu.VMEM((_W + 16,), jnp.int32),
        pltpu.VMEM((7 * _NP,), jnp.float32),
        pltpu.SemaphoreType.DMA,
        pltpu.SemaphoreType.DMA,
        pltpu.SemaphoreType.DMA,
        pltpu.SemaphoreType.DMA,
        pltpu.SemaphoreType.DMA,
    ],
    compiler_params=pltpu.CompilerParams(needs_layout_passes=False),
)(_p2_body)


def kernel(boxes, scores):
    order = jnp.argsort(-scores)
    b = jnp.take(boxes, order, axis=0)
    s = jnp.take(scores, order)
    c = b[:, :3]
    sz = b[:, 3:]
    lo = c - sz / 2
    hi = c + sz / 2
    vol = jnp.prod(sz, axis=-1)
    pad = _NP - _N
    far = jnp.full((pad,), 1e9, jnp.float32)
    zpad = jnp.zeros((pad,), jnp.float32)

    def padcat(x, p):
        return jnp.concatenate([x, p])

    p7 = jnp.concatenate([
        padcat(lo[:, 0], far), padcat(lo[:, 1], far), padcat(lo[:, 2], far),
        padcat(hi[:, 0], far), padcat(hi[:, 1], far), padcat(hi[:, 2], far),
        padcat(vol, zpad),
    ])
    o7 = jnp.concatenate([
        padcat(s, zpad),
        padcat(b[:, 0], zpad), padcat(b[:, 1], zpad), padcat(b[:, 2], zpad),
        padcat(b[:, 3], zpad), padcat(b[:, 4], zpad), padcat(b[:, 5], zpad),
    ])
    m = _phase1(p7)
    out7 = _phase2(m, o7)
    return out7.reshape(7, _NP).T[:_N]


# speedup vs baseline: 1.4115x; 1.0026x over previous
import functools

import jax
import jax.numpy as jnp
from jax import lax
from jax.experimental import pallas as pl
from jax.experimental.pallas import tpu as pltpu
from jax.experimental.pallas import tpu_sc as plsc

_N = 5000
_NP = 5120
_W = _NP // 32
_NC = 2
_NS = 16
_NW = _NC * _NS
_CHUNK_ROWS = 16
_NCHUNKS = _NP // _CHUNK_ROWS
_T = _NCHUNKS // _NW
_R2 = 64
_IOU = 0.25

_mesh = plsc.VectorSubcoreMesh(core_axis_name="c", subcore_axis_name="s")


_NG = _NP // 32
_GPW = _NG // _NW


def _p1_body(p7u_hbm, ord_hbm, m_hbm, p7u_v, ord_v, p7_v, buf_v):
    cax = lax.axis_index("c")
    sax = lax.axis_index("s")
    wid = sax * _NC + cax
    pltpu.sync_copy(p7u_hbm, p7u_v)
    pltpu.sync_copy(ord_hbm, ord_v)

    def sort_gather(a, _):
        def gv(v, __):
            idx = ord_v[pl.ds(v * 16, 16)]
            p7_v[pl.ds(a * _NP + v * 16, 16)] = plsc.load_gather(
                p7u_v, [idx + a * _NP])
            return __
        lax.fori_loop(0, _NP // 16, gv, 0)
        return _

    lax.fori_loop(0, 7, sort_gather, 0)
    l16 = lax.iota(jnp.int32, 16)
    l160 = l16 * _W
    zz = jnp.zeros((16,), jnp.int32)

    def group_body(t, _):
        g_fwd = t * _NW + wid
        g_rev = (t + 1) * _NW - 1 - wid
        g = jnp.where(t % 2 == 0, g_fwd, g_rev)
        c0 = g * 32
        jlA = c0 + l16
        jlB = c0 + 16 + l16
        jA = [p7_v[pl.ds(a * _NP + c0, 16)] for a in range(7)]
        jB = [p7_v[pl.ds(a * _NP + c0 + 16, 16)] for a in range(7)]
        for q in range(32 * _W // 16):
            buf_v[pl.ds(q * 16, 16)] = zz

        def word_eval(w, masked):
            accA = jnp.zeros((16,), jnp.int32)
            accB = jnp.zeros((16,), jnp.int32)
            for h in range(2):
                base = w * 32 + h * 16
                vecs = [p7_v[pl.ds(a * _NP + base, 16)] for a in range(7)]
                for dl in range(16):
                    di = h * 16 + dl
                    i = base + dl
                    b7 = [vecs[a][dl] for a in range(7)]
                    bitval = jnp.int32(-2147483648) if di == 31 else jnp.int32(1 << di)
                    for side in range(2):
                        jv = jA if side == 0 else jB
                        jl = jlA if side == 0 else jlB
                        dz = jnp.maximum(
                            jnp.minimum(jv[3], b7[3]) - jnp.maximum(jv[0], b7[0]), 0.0)
                        dy = jnp.maximum(
                            jnp.minimum(jv[4], b7[4]) - jnp.maximum(jv[1], b7[1]), 0.0)
                        dx = jnp.maximum(
                            jnp.minimum(jv[5], b7[5]) - jnp.maximum(jv[2], b7[2]), 0.0)
                        ov = (dz * dy) * dx
                        un = (b7[6] + jv[6]) - ov
                        hit = ov > _IOU * un
                        if masked:
                            hit = hit & (i > jl)
                        bits = jnp.where(hit, bitval, jnp.int32(0))
                        if side == 0:
                            accA = accA | bits
                        else:
                            accB = accB | bits
            plsc.store_scatter(buf_v, [l160 + w], accA)
            plsc.store_scatter(buf_v, [l160 + 16 * _W + w], accB)

        word_eval(g, True)

        def wbody(w, carry):
            word_eval(w, False)
            return carry

        lax.fori_loop(g + 1, _W, wbody, 0)
        pltpu.sync_copy(buf_v, m_hbm.at[pl.ds(c0 * _W, 32 * _W)])
        return _

    lax.fori_loop(0, _GPW, group_body, 0)


def _p2_body(m_hbm, o7u_hbm, ord_hbm, out_hbm, mbuf_v, rem_v, io_v, iou_v,
             ord_v, s0, s1, s2, s3, s4, s5):
    cax = lax.axis_index("c")
    sax = lax.axis_index("s")
    wid = sax * _NC + cax
    l16 = lax.iota(jnp.int32, 16)
    sems = (s0, s1, s2, s3)
    _CH = 32
    _CHW = _CH * _W
    _NB = 4
    _NCH = _NP // _CH

    @pl.when(wid == 0)
    def _():
        io_cp = pltpu.async_copy(o7u_hbm, iou_v, s4)
        ord_cp = pltpu.async_copy(ord_hbm, ord_v, s5)
        zv = jnp.zeros((16,), jnp.int32)
        for k in range(11):
            rem_v[pl.ds(k * 16, 16)] = zv
        for b in range(_NB):
            pltpu.async_copy(
                m_hbm.at[pl.ds(b * _CHW, _CHW)],
                mbuf_v.at[pl.ds(b * _CHW, _CHW)],
                sems[b],
            )

        def super_body(q, carry):
            for b in range(_NB):
                cc = q * _NB + b
                pltpu.make_async_copy(
                    m_hbm.at[pl.ds(0, _CHW)],
                    mbuf_v.at[pl.ds(b * _CHW, _CHW)],
                    sems[b],
                ).wait()
                z16 = jnp.zeros((16,), jnp.int32)
                localv = plsc.load_gather(rem_v, [z16 + cc])
                for r in range(_CH):
                    dwv = plsc.load_gather(
                        mbuf_v, [z16 + (b * _CHW + r * _W + cc)])
                    bitc = jnp.int32(-2147483648) if r == 31 else jnp.int32(1 << r)
                    imsk = jnp.where((localv & bitc) == 0, jnp.int32(-1), jnp.int32(0))
                    localv = localv | (dwv & imsk)
                for half in range(4):
                    msks = []
                    for rr in range(8):
                        r = half * 8 + rr
                        bitc = (jnp.int32(-2147483648) if r == 31
                                else jnp.int32(1 << r))
                        msks.append((localv & bitc) == 0)
                    for k in range(10):
                        acc = rem_v[pl.ds(k * 16, 16)]
                        for rr in range(8):
                            r = half * 8 + rr
                            mv = mbuf_v[pl.ds(b * _CHW + r * _W + k * 16, 16)]
                            acc = acc | jnp.where(msks[rr], mv, zv)
                        rem_v[pl.ds(k * 16, 16)] = acc
                    del msks

                @pl.when(cc + _NB < _NCH)
                def _():
                    pltpu.async_copy(
                        m_hbm.at[pl.ds((cc + _NB) * _CHW, _CHW)],
                        mbuf_v.at[pl.ds(b * _CHW, _CHW)],
                        sems[b],
                    )

            return carry

        lax.fori_loop(0, _NCH // _NB, super_body, 0)
        io_cp.wait()
        ord_cp.wait()

        def mask_body(v, carry):
            word = rem_v[pl.ds(v // 2, 16)][0]
            sh = l16 + (v & 1) * 16
            bits = (word >> sh) & 1
            keep = 1.0 - bits.astype(jnp.float32)
            idx = ord_v[pl.ds(v * 16, 16)]
            for a in range(7):
                vals = plsc.load_gather(iou_v, [idx + a * _NP])
                io_v[pl.ds(a * _NP + v * 16, 16)] = vals * keep
            return carry

        lax.fori_loop(0, _NP // 16, mask_body, 0)
        pltpu.sync_copy(io_v, out_hbm)


_phase1 = functools.partial(
    pl.kernel,
    out_type=jax.ShapeDtypeStruct((_NP * _W,), jnp.int32),
    mesh=_mesh,
    scratch_types=[
        pltpu.VMEM((7 * _NP,), jnp.float32),
        pltpu.VMEM((_NP,), jnp.int32),
        pltpu.VMEM((7 * _NP,), jnp.float32),
        pltpu.VMEM((32 * _W,), jnp.int32),
    ],
    compiler_params=pltpu.CompilerParams(needs_layout_passes=False),
)(_p1_body)

_phase2 = functools.partial(
    pl.kernel,
    out_type=jax.ShapeDtypeStruct((7 * _NP,), jnp.float32),
    mesh=_mesh,
    scratch_types=[
        pltpu.VMEM((4 * 32 * _W + 192,), jnp.int32),
        pltpu.VMEM((_W + 16,), jnp.int32),
        pltpu.VMEM((7 * _NP,), jnp.float32),
        pltpu.VMEM((7 * _NP,), jnp.float32),
        pltpu.VMEM((_NP,), jnp.int32),
        pltpu.SemaphoreType.DMA,
        pltpu.SemaphoreType.DMA,
        pltpu.SemaphoreType.DMA,
        pltpu.SemaphoreType.DMA,
        pltpu.SemaphoreType.DMA,
        pltpu.SemaphoreType.DMA,
    ],
    compiler_params=pltpu.CompilerParams(needs_layout_passes=False),
)(_p2_body)


def kernel(boxes, scores):
    order = jnp.argsort(-scores)
    ordp = jnp.concatenate(
        [order.astype(jnp.int32), jnp.arange(_N, _NP, dtype=jnp.int32)])
    c = boxes[:, :3]
    sz = boxes[:, 3:]
    lo = c - sz / 2
    hi = c + sz / 2
    vol = jnp.prod(sz, axis=-1)
    pad = _NP - _N
    far = jnp.full((pad,), 1e9, jnp.float32)
    zpad = jnp.zeros((pad,), jnp.float32)

    def padcat(x, p):
        return jnp.concatenate([x, p])

    p7u = jnp.concatenate([
        padcat(lo[:, 0], far), padcat(lo[:, 1], far), padcat(lo[:, 2], far),
        padcat(hi[:, 0], far), padcat(hi[:, 1], far), padcat(hi[:, 2], far),
        padcat(vol, zpad),
    ])
    o7u = jnp.concatenate([
        padcat(scores, zpad),
        padcat(boxes[:, 0], zpad), padcat(boxes[:, 1], zpad),
        padcat(boxes[:, 2], zpad), padcat(boxes[:, 3], zpad),
        padcat(boxes[:, 4], zpad), padcat(boxes[:, 5], zpad),
    ])
    m = _phase1(p7u, ordp)
    out7 = _phase2(m, o7u, ordp)
    return out7.reshape(7, _NP).T[:_N]


# speedup vs baseline: 1.4116x; 1.0000x over previous
import functools

import jax
import jax.numpy as jnp
from jax import lax
from jax.experimental import pallas as pl
from jax.experimental.pallas import tpu as pltpu
from jax.experimental.pallas import tpu_sc as plsc

_N = 5000
_NP = 5120
_W = _NP // 32
_NC = 2
_NS = 16
_NW = _NC * _NS
_IOU = 0.25

_mesh = plsc.VectorSubcoreMesh(core_axis_name="c", subcore_axis_name="s")


_NG = _NP // 32
_GPW = _NG // _NW


def _p1_body(p7u_hbm, ord_hbm, m_hbm, p7u_v, ord_v, p7_v, buf_v):
    cax = lax.axis_index("c")
    sax = lax.axis_index("s")
    wid = sax * _NC + cax
    pltpu.sync_copy(p7u_hbm, p7u_v)
    pltpu.sync_copy(ord_hbm, ord_v)

    def sort_gather(a, _):
        def gv(v, __):
            idx = ord_v[pl.ds(v * 16, 16)]
            p7_v[pl.ds(a * _NP + v * 16, 16)] = plsc.load_gather(
                p7u_v, [idx + a * _NP])
            return __
        lax.fori_loop(0, _NP // 16, gv, 0)
        return _

    lax.fori_loop(0, 7, sort_gather, 0)
    l16 = lax.iota(jnp.int32, 16)
    l160 = l16 * _W
    zz = jnp.zeros((16,), jnp.int32)

    def group_body(t, _):
        g_fwd = t * _NW + wid
        g_rev = (t + 1) * _NW - 1 - wid
        g = jnp.where(t % 2 == 0, g_fwd, g_rev)
        c0 = g * 32
        jlA = c0 + l16
        jlB = c0 + 16 + l16
        jA = [p7_v[pl.ds(a * _NP + c0, 16)] for a in range(7)]
        jB = [p7_v[pl.ds(a * _NP + c0 + 16, 16)] for a in range(7)]
        for q in range(32 * _W // 16):
            buf_v[pl.ds(q * 16, 16)] = zz

        def word_eval(w, masked):
            accA = jnp.zeros((16,), jnp.int32)
            accB = jnp.zeros((16,), jnp.int32)
            for h in range(2):
                base = w * 32 + h * 16
                vecs = [p7_v[pl.ds(a * _NP + base, 16)] for a in range(7)]
                for dl in range(16):
                    di = h * 16 + dl
                    i = base + dl
                    b7 = [vecs[a][dl] for a in range(7)]
                    bitval = jnp.int32(-2147483648) if di == 31 else jnp.int32(1 << di)
                    for side in range(2):
                        jv = jA if side == 0 else jB
                        jl = jlA if side == 0 else jlB
                        dz = jnp.maximum(
                            jnp.minimum(jv[3], b7[3]) - jnp.maximum(jv[0], b7[0]), 0.0)
                        dy = jnp.maximum(
                            jnp.minimum(jv[4], b7[4]) - jnp.maximum(jv[1], b7[1]), 0.0)
                        dx = jnp.maximum(
                            jnp.minimum(jv[5], b7[5]) - jnp.maximum(jv[2], b7[2]), 0.0)
                        ov = (dz * dy) * dx
                        un = (b7[6] + jv[6]) - ov
                        hit = ov > _IOU * un
                        if masked:
                            hit = hit & (i > jl)
                        bits = jnp.where(hit, bitval, jnp.int32(0))
                        if side == 0:
                            accA = accA | bits
                        else:
                            accB = accB | bits
            plsc.store_scatter(buf_v, [l160 + w], accA)
            plsc.store_scatter(buf_v, [l160 + 16 * _W + w], accB)

        word_eval(g, True)

        def wbody(w, carry):
            word_eval(w, False)
            return carry

        lax.fori_loop(g + 1, _W, wbody, 0)
        pltpu.sync_copy(buf_v, m_hbm.at[pl.ds(c0 * _W, 32 * _W)])
        return _

    lax.fori_loop(0, _GPW, group_body, 0)


def _p2_body(m_hbm, o7u_hbm, ord_hbm, out_hbm, mbuf_v, rem_v, io_v, iou_v,
             ord_v, s0, s1, s2, s3, s4, s5):
    cax = lax.axis_index("c")
    sax = lax.axis_index("s")
    wid = sax * _NC + cax
    l16 = lax.iota(jnp.int32, 16)
    sems = (s0, s1, s2, s3)
    _CH = 32
    _CHW = _CH * _W
    _NB = 4
    _NCH = _NP // _CH

    @pl.when(wid == 0)
    def _():
        io_cp = pltpu.async_copy(o7u_hbm, iou_v, s4)
        ord_cp = pltpu.async_copy(ord_hbm, ord_v, s5)
        zv = jnp.zeros((16,), jnp.int32)
        for k in range(11):
            rem_v[pl.ds(k * 16, 16)] = zv
        for b in range(_NB):
            pltpu.async_copy(
                m_hbm.at[pl.ds(b * _CHW, _CHW)],
                mbuf_v.at[pl.ds(b * _CHW, _CHW)],
                sems[b],
            )

        def super_body(q, carry):
            for b in range(_NB):
                cc = q * _NB + b
                pltpu.make_async_copy(
                    m_hbm.at[pl.ds(0, _CHW)],
                    mbuf_v.at[pl.ds(b * _CHW, _CHW)],
                    sems[b],
                ).wait()
                z16 = jnp.zeros((16,), jnp.int32)
                localv = plsc.load_gather(rem_v, [z16 + cc])
                for r in range(_CH):
                    dwv = plsc.load_gather(
                        mbuf_v, [z16 + (b * _CHW + r * _W + cc)])
                    bitc = jnp.int32(-2147483648) if r == 31 else jnp.int32(1 << r)
                    imsk = jnp.where((localv & bitc) == 0, jnp.int32(-1), jnp.int32(0))
                    localv = localv | (dwv & imsk)
                for half in range(4):
                    msks = []
                    for rr in range(8):
                        r = half * 8 + rr
                        bitc = (jnp.int32(-2147483648) if r == 31
                                else jnp.int32(1 << r))
                        msks.append((localv & bitc) == 0)
                    for k in range(10):
                        acc = rem_v[pl.ds(k * 16, 16)]
                        for rr in range(8):
                            r = half * 8 + rr
                            mv = mbuf_v[pl.ds(b * _CHW + r * _W + k * 16, 16)]
                            acc = acc | jnp.where(msks[rr], mv, zv)
                        rem_v[pl.ds(k * 16, 16)] = acc
                    del msks

                @pl.when(cc + _NB < _NCH)
                def _():
                    pltpu.async_copy(
                        m_hbm.at[pl.ds((cc + _NB) * _CHW, _CHW)],
                        mbuf_v.at[pl.ds(b * _CHW, _CHW)],
                        sems[b],
                    )

            return carry

        lax.fori_loop(0, _NCH // _NB, super_body, 0)
        io_cp.wait()
        ord_cp.wait()

        def mask_body(v, carry):
            word = rem_v[pl.ds(v // 2, 16)][0]
            sh = l16 + (v & 1) * 16
            bits = (word >> sh) & 1
            keep = 1.0 - bits.astype(jnp.float32)
            idx = ord_v[pl.ds(v * 16, 16)]
            for a in range(7):
                vals = plsc.load_gather(iou_v, [idx + a * _NP])
                io_v[pl.ds(a * _NP + v * 16, 16)] = vals * keep
            return carry

        lax.fori_loop(0, _NP // 16, mask_body, 0)
        pltpu.sync_copy(io_v, out_hbm)


_phase1 = functools.partial(
    pl.kernel,
    out_type=jax.ShapeDtypeStruct((_NP * _W,), jnp.int32),
    mesh=_mesh,
    scratch_types=[
        pltpu.VMEM((7 * _NP,), jnp.float32),
        pltpu.VMEM((_NP,), jnp.int32),
        pltpu.VMEM((7 * _NP,), jnp.float32),
        pltpu.VMEM((32 * _W,), jnp.int32),
    ],
    compiler_params=pltpu.CompilerParams(needs_layout_passes=False),
)(_p1_body)

_phase2 = functools.partial(
    pl.kernel,
    out_type=jax.ShapeDtypeStruct((7 * _NP,), jnp.float32),
    mesh=_mesh,
    scratch_types=[
        pltpu.VMEM((4 * 32 * _W + 192,), jnp.int32),
        pltpu.VMEM((_W + 16,), jnp.int32),
        pltpu.VMEM((7 * _NP,), jnp.float32),
        pltpu.VMEM((7 * _NP,), jnp.float32),
        pltpu.VMEM((_NP,), jnp.int32),
        pltpu.SemaphoreType.DMA,
        pltpu.SemaphoreType.DMA,
        pltpu.SemaphoreType.DMA,
        pltpu.SemaphoreType.DMA,
        pltpu.SemaphoreType.DMA,
        pltpu.SemaphoreType.DMA,
    ],
    compiler_params=pltpu.CompilerParams(needs_layout_passes=False),
)(_p2_body)


def kernel(boxes, scores):
    order = jnp.argsort(-scores)
    ordp = jnp.concatenate(
        [order.astype(jnp.int32), jnp.arange(_N, _NP, dtype=jnp.int32)])
    c = boxes[:, :3]
    sz = boxes[:, 3:]
    lo = c - sz / 2
    hi = c + sz / 2
    vol = jnp.prod(sz, axis=-1)
    pad = _NP - _N
    far = jnp.full((pad,), 1e9, jnp.float32)
    zpad = jnp.zeros((pad,), jnp.float32)

    def padcat(x, p):
        return jnp.concatenate([x, p])

    p7u = jnp.concatenate([
        padcat(lo[:, 0], far), padcat(lo[:, 1], far), padcat(lo[:, 2], far),
        padcat(hi[:, 0], far), padcat(hi[:, 1], far), padcat(hi[:, 2], far),
        padcat(vol, zpad),
    ])
    o7u = jnp.concatenate([
        padcat(scores, zpad),
        padcat(boxes[:, 0], zpad), padcat(boxes[:, 1], zpad),
        padcat(boxes[:, 2], zpad), padcat(boxes[:, 3], zpad),
        padcat(boxes[:, 4], zpad), padcat(boxes[:, 5], zpad),
    ])
    m = _phase1(p7u, ordp)
    out7 = _phase2(m, o7u, ordp)
    return out7.reshape(7, _NP).T[:_N]
